# baseline (speedup 1.0000x reference)
import functools

import jax
import jax.numpy as jnp
from jax import lax
from jax.experimental import pallas as pl
from jax.experimental.pallas import tpu as pltpu
from jax.experimental.pallas import tpu_sc as plsc

N = 10000
E = 320000
D_IN = 128
H1 = 64
H2 = 32
EPS = 1e-5

NC = 2
NS = 16
NW = NC * NS
EPW = E // NW
W = 128
NWIN = 80
EPWP = NWIN * W
EP = NW * EPWP
NP = 10240
NPT = NP // NS
PAD_DST = NP - 8
CW = 16
ZCH = 128


def _proj_body(x_ref, wl_ref, wr_ref, yl_ref, yr_ref):
    x = x_ref[...]
    yl_ref[...] = lax.dot_general(
        x, wl_ref[...], (((1,), (1,)), ((), ())),
        preferred_element_type=jnp.float32,
        precision=lax.Precision.HIGHEST)
    yr_ref[...] = lax.dot_general(
        x, wr_ref[...], (((1,), (1,)), ((), ())),
        preferred_element_type=jnp.float32,
        precision=lax.Precision.HIGHEST)


def _proj(x, wl, wr):
    h = wl.shape[0]
    return pl.pallas_call(
        _proj_body,
        out_shape=(
            jax.ShapeDtypeStruct((x.shape[0], h), jnp.float32),
            jax.ShapeDtypeStruct((x.shape[0], h), jnp.float32),
        ),
    )(x, wl, wr)


def _bn_relu(h, g_ref, be_ref):
    mu = jnp.mean(h, axis=0, keepdims=True)
    var = jnp.mean((h - mu) * (h - mu), axis=0, keepdims=True)
    hn = (h - mu) * lax.rsqrt(var + EPS) * g_ref[...] + be_ref[...]
    return jnp.maximum(hn, 0.0)


def _mid_body(acc_ref, cnt_ref, r_ref, b_ref, g_ref, be_ref, wl_ref, wr_ref,
              y2_ref, r2_ref):
    cnt = cnt_ref[0, 0:N, 0:1] + cnt_ref[1, 0:N, 0:1]
    agg = (acc_ref[0, 0:N] + acc_ref[1, 0:N]) / jnp.maximum(cnt, 1.0)
    h = agg + b_ref[...] + r_ref[...]
    h = _bn_relu(h, g_ref, be_ref)
    y2_ref[...] = lax.dot_general(
        h, wl_ref[...], (((1,), (1,)), ((), ())),
        preferred_element_type=jnp.float32,
        precision=lax.Precision.HIGHEST)
    r2_ref[...] = lax.dot_general(
        h, wr_ref[...], (((1,), (1,)), ((), ())),
        preferred_element_type=jnp.float32,
        precision=lax.Precision.HIGHEST)


def _mid(acc, cnt, r, b, g, be, wl, wr):
    return pl.pallas_call(
        _mid_body,
        out_shape=(
            jax.ShapeDtypeStruct((N, H2), jnp.float32),
            jax.ShapeDtypeStruct((N, H2), jnp.float32),
        ),
        compiler_params=pltpu.CompilerParams(vmem_limit_bytes=64 * 1024 * 1024),
    )(acc, cnt, r, b, g, be, wl, wr)


def _fin_body(acc_ref, cnt_ref, r_ref, b_ref, g_ref, be_ref, o_ref):
    cnt = cnt_ref[0, 0:N, 0:1] + cnt_ref[1, 0:N, 0:1]
    agg = (acc_ref[0, 0:N] + acc_ref[1, 0:N]) / jnp.maximum(cnt, 1.0)
    h = agg + b_ref[...] + r_ref[...]
    o_ref[...] = _bn_relu(h, g_ref, be_ref)


def _fin(acc, cnt, r, b, g, be):
    return pl.pallas_call(
        _fin_body,
        out_shape=jax.ShapeDtypeStruct((N, H2), jnp.float32),
    )(acc, cnt, r, b, g, be)


def _sc_agg_body(d, with_cnt, *refs):
    if with_cnt:
        (y_hbm, src_hbm, dst_hbm, acc_out, cnt_out,
         idx_s, idx_d, rows_a, rows_b, ones_v, zer_v, zerc_v,
         acc_sh, cnt_sh, sem, sem_a, sem_b) = refs
    else:
        (y_hbm, src_hbm, dst_hbm, acc_out,
         idx_s, idx_d, rows_a, rows_b, zer_v, acc_sh, sem, sem_a, sem_b) = refs

    c = lax.axis_index("c")
    s = lax.axis_index("s")
    wid = s * NC + c

    @pl.loop(0, ZCH)
    def _(i):
        for k in range(d // 16):
            zer_v[i, pl.ds(16 * k, 16)] = jnp.zeros((16,), jnp.float32)
        if with_cnt:
            zerc_v[i] = jnp.zeros((16,), jnp.float32)

    if with_cnt:
        @pl.loop(0, W)
        def _(i):
            ones_v[i] = jnp.ones((16,), jnp.float32)

    cp_s = pltpu.async_copy(src_hbm.at[wid], idx_s, sem)
    cp_d = pltpu.async_copy(dst_hbm.at[wid], idx_d, sem)

    @pl.loop(0, NPT // ZCH)
    def _(k):
        base = s * NPT + k * ZCH
        pltpu.sync_copy(zer_v, acc_sh.at[pl.ds(base, ZCH)])
        if with_cnt:
            pltpu.sync_copy(zerc_v, cnt_sh.at[pl.ds(base, ZCH)])

    cp_s.wait()
    cp_d.wait()
    plsc.subcore_barrier()

    @pl.loop(0, NWIN)
    def _(j):
        cp = pltpu.async_copy(y_hbm.at[idx_s.at[j]], rows_a, sem_a)
        if with_cnt:
            pltpu.sync_copy(ones_v, cnt_sh.at[idx_d.at[j]], add=True)
        cp.wait()
        pltpu.sync_copy(rows_a, acc_sh.at[idx_d.at[j]], add=True)

    plsc.subcore_barrier()

    pltpu.sync_copy(acc_sh.at[pl.ds(s * NPT, NPT)],
                    acc_out.at[c, pl.ds(s * NPT, NPT)])
    if with_cnt:
        pltpu.sync_copy(cnt_sh.at[pl.ds(s * NPT, NPT)],
                        cnt_out.at[c, pl.ds(s * NPT, NPT)])


def _make_sc_agg(d, with_cnt):
    mesh = plsc.VectorSubcoreMesh(core_axis_name="c", subcore_axis_name="s")
    out_type = [jax.ShapeDtypeStruct((NC, NP, d), jnp.float32)]
    scratch = [
        pltpu.VMEM((NWIN, W), jnp.int32),
        pltpu.VMEM((NWIN, W), jnp.int32),
        pltpu.VMEM((W, d), jnp.float32),
        pltpu.VMEM((W, d), jnp.float32),
    ]
    if with_cnt:
        out_type.append(jax.ShapeDtypeStruct((NC, NP, CW), jnp.float32))
        scratch.append(pltpu.VMEM((W, CW), jnp.float32))
    scratch.append(pltpu.VMEM((ZCH, d), jnp.float32))
    if with_cnt:
        scratch.append(pltpu.VMEM((ZCH, CW), jnp.float32))
    scratch.append(pltpu.VMEM_SHARED((NP, d), jnp.float32))
    if with_cnt:
        scratch.append(pltpu.VMEM_SHARED((NP, CW), jnp.float32))
    scratch.append(pltpu.SemaphoreType.DMA)
    scratch.append(pltpu.SemaphoreType.DMA)
    scratch.append(pltpu.SemaphoreType.DMA)
    return pl.kernel(
        functools.partial(_sc_agg_body, d, with_cnt),
        out_type=out_type,
        mesh=mesh,
        scratch_types=scratch,
        compiler_params=pltpu.CompilerParams(use_tc_tiling_on_sc=False),
    )


def kernel(x, edge_index, W1l, b1, W1r, g1, be1, W2l, b2, W2r, g2, be2):
    ppw = EPWP - E // NW
    pad_src = jnp.zeros((NW, ppw), jnp.int32)
    pad_dst = jnp.broadcast_to(N + jnp.arange(ppw, dtype=jnp.int32),
                               (NW, ppw))
    src = jnp.concatenate(
        [edge_index[0].reshape(NW, E // NW), pad_src], axis=1).reshape(
            NW, NWIN, W)
    dst = jnp.concatenate(
        [edge_index[1].reshape(NW, E // NW), pad_dst], axis=1).reshape(
            NW, NWIN, W)

    y1, r1 = _proj(x, W1l, W1r)
    acc1, cnt = _make_sc_agg(H1, True)(y1, src, dst)
    y2, r2 = _mid(acc1, cnt, r1, b1.reshape(1, H1), g1.reshape(1, H1),
                  be1.reshape(1, H1), W2l, W2r)
    acc2, = _make_sc_agg(H2, False)(y2, src, dst)
    return _fin(acc2, cnt, r2, b2.reshape(1, H2), g2.reshape(1, H2),
                be2.reshape(1, H2))


# speedup vs baseline: 2.0302x; 2.0302x over previous
import functools

import jax
import jax.numpy as jnp
from jax import lax
from jax.experimental import pallas as pl
from jax.experimental.pallas import tpu as pltpu
from jax.experimental.pallas import tpu_sc as plsc

N = 10000
E = 320000
D_IN = 128
H1 = 64
H2 = 32
EPS = 1e-5

NC = 2
NS = 16
NW = NC * NS
EPW = E // NW
W = 80
NWIN = EPW // W
NP = 10240
NPT = NP // NS
CW = 16
ZCH = 128


def _proj_body(x_ref, wl_ref, wr_ref, yl_ref, yr_ref):
    x = x_ref[...]
    yl_ref[...] = lax.dot_general(
        x, wl_ref[...], (((1,), (1,)), ((), ())),
        preferred_element_type=jnp.float32,
        precision=lax.Precision.HIGHEST)
    yr_ref[...] = lax.dot_general(
        x, wr_ref[...], (((1,), (1,)), ((), ())),
        preferred_element_type=jnp.float32,
        precision=lax.Precision.HIGHEST)


def _proj(x, wl, wr):
    h = wl.shape[0]
    return pl.pallas_call(
        _proj_body,
        out_shape=(
            jax.ShapeDtypeStruct((x.shape[0], h), jnp.float32),
            jax.ShapeDtypeStruct((x.shape[0], h), jnp.float32),
        ),
    )(x, wl, wr)


def _bn_relu(h, g_ref, be_ref):
    mu = jnp.mean(h, axis=0, keepdims=True)
    var = jnp.mean((h - mu) * (h - mu), axis=0, keepdims=True)
    hn = (h - mu) * lax.rsqrt(var + EPS) * g_ref[...] + be_ref[...]
    return jnp.maximum(hn, 0.0)


def _mid_body(acc_ref, cnt_ref, r_ref, b_ref, g_ref, be_ref, wl_ref, wr_ref,
              y2_ref, r2_ref):
    cnt = cnt_ref[0, 0:N, 0:1] + cnt_ref[1, 0:N, 0:1]
    agg = (acc_ref[0, 0:N] + acc_ref[1, 0:N]) / jnp.maximum(cnt, 1.0)
    h = agg + b_ref[...] + r_ref[...]
    h = _bn_relu(h, g_ref, be_ref)
    y2_ref[...] = lax.dot_general(
        h, wl_ref[...], (((1,), (1,)), ((), ())),
        preferred_element_type=jnp.float32,
        precision=lax.Precision.HIGHEST)
    r2_ref[...] = lax.dot_general(
        h, wr_ref[...], (((1,), (1,)), ((), ())),
        preferred_element_type=jnp.float32,
        precision=lax.Precision.HIGHEST)


def _mid(acc, cnt, r, b, g, be, wl, wr):
    return pl.pallas_call(
        _mid_body,
        out_shape=(
            jax.ShapeDtypeStruct((N, H2), jnp.float32),
            jax.ShapeDtypeStruct((N, H2), jnp.float32),
        ),
        compiler_params=pltpu.CompilerParams(vmem_limit_bytes=64 * 1024 * 1024),
    )(acc, cnt, r, b, g, be, wl, wr)


def _fin_body(acc_ref, cnt_ref, r_ref, b_ref, g_ref, be_ref, o_ref):
    cnt = cnt_ref[0, 0:N, 0:1] + cnt_ref[1, 0:N, 0:1]
    agg = (acc_ref[0, 0:N] + acc_ref[1, 0:N]) / jnp.maximum(cnt, 1.0)
    h = agg + b_ref[...] + r_ref[...]
    o_ref[...] = _bn_relu(h, g_ref, be_ref)


def _fin(acc, cnt, r, b, g, be):
    return pl.pallas_call(
        _fin_body,
        out_shape=jax.ShapeDtypeStruct((N, H2), jnp.float32),
    )(acc, cnt, r, b, g, be)


def _sc_agg_body(d, with_cnt, *refs):
    if with_cnt:
        (y_hbm, src_hbm, dst_hbm, acc_out, cnt_out,
         idx_s, idx_d, rows_a, rows_b, ones_v, zer_v, zerc_v,
         acc_sh, cnt_sh, sem, sem_a, sem_b) = refs
    else:
        (y_hbm, src_hbm, dst_hbm, acc_out,
         idx_s, idx_d, rows_a, rows_b, zer_v, acc_sh, sem, sem_a, sem_b) = refs

    c = lax.axis_index("c")
    s = lax.axis_index("s")
    wid = s * NC + c

    @pl.loop(0, ZCH)
    def _(i):
        for k in range(d // 16):
            zer_v[i, pl.ds(16 * k, 16)] = jnp.zeros((16,), jnp.float32)
        if with_cnt:
            zerc_v[i] = jnp.zeros((16,), jnp.float32)

    if with_cnt:
        @pl.loop(0, W)
        def _(i):
            ones_v[i] = jnp.ones((16,), jnp.float32)

    cp_s = pltpu.async_copy(src_hbm.at[wid], idx_s, sem)
    cp_d = pltpu.async_copy(dst_hbm.at[wid], idx_d, sem)

    @pl.loop(0, NPT // ZCH)
    def _(k):
        base = s * NPT + k * ZCH
        pltpu.sync_copy(zer_v, acc_sh.at[pl.ds(base, ZCH)])
        if with_cnt:
            pltpu.sync_copy(zerc_v, cnt_sh.at[pl.ds(base, ZCH)])

    cp_s.wait()
    cp_d.wait()
    plsc.subcore_barrier()

    pltpu.async_copy(y_hbm.at[idx_s.at[0]], rows_a, sem_a)

    @pl.loop(0, NWIN - 1, step=2)
    def _(j):
        pltpu.async_copy(y_hbm.at[idx_s.at[j + 1]], rows_b, sem_b)
        if with_cnt:
            pltpu.sync_copy(ones_v, cnt_sh.at[idx_d.at[j]], add=True)
        pltpu.make_async_copy(y_hbm.at[pl.ds(0, W)], rows_a, sem_a).wait()
        pltpu.sync_copy(rows_a, acc_sh.at[idx_d.at[j]], add=True)

        @pl.when(j + 2 < NWIN)
        def _():
            pltpu.async_copy(y_hbm.at[idx_s.at[j + 2]], rows_a, sem_a)

        if with_cnt:
            pltpu.sync_copy(ones_v, cnt_sh.at[idx_d.at[j + 1]], add=True)
        pltpu.make_async_copy(y_hbm.at[pl.ds(0, W)], rows_b, sem_b).wait()
        pltpu.sync_copy(rows_b, acc_sh.at[idx_d.at[j + 1]], add=True)

    if NWIN % 2:
        if with_cnt:
            pltpu.sync_copy(ones_v, cnt_sh.at[idx_d.at[NWIN - 1]], add=True)
        pltpu.make_async_copy(y_hbm.at[pl.ds(0, W)], rows_a, sem_a).wait()
        pltpu.sync_copy(rows_a, acc_sh.at[idx_d.at[NWIN - 1]], add=True)

    plsc.subcore_barrier()

    pltpu.sync_copy(acc_sh.at[pl.ds(s * NPT, NPT)],
                    acc_out.at[c, pl.ds(s * NPT, NPT)])
    if with_cnt:
        pltpu.sync_copy(cnt_sh.at[pl.ds(s * NPT, NPT)],
                        cnt_out.at[c, pl.ds(s * NPT, NPT)])


def _make_sc_agg(d, with_cnt):
    mesh = plsc.VectorSubcoreMesh(core_axis_name="c", subcore_axis_name="s")
    out_type = [jax.ShapeDtypeStruct((NC, NP, d), jnp.float32)]
    scratch = [
        pltpu.VMEM((NWIN, W), jnp.int32),
        pltpu.VMEM((NWIN, W), jnp.int32),
        pltpu.VMEM((W, d), jnp.float32),
        pltpu.VMEM((W, d), jnp.float32),
    ]
    if with_cnt:
        out_type.append(jax.ShapeDtypeStruct((NC, NP, CW), jnp.float32))
        scratch.append(pltpu.VMEM((W, CW), jnp.float32))
    scratch.append(pltpu.VMEM((ZCH, d), jnp.float32))
    if with_cnt:
        scratch.append(pltpu.VMEM((ZCH, CW), jnp.float32))
    scratch.append(pltpu.VMEM_SHARED((NP, d), jnp.float32))
    if with_cnt:
        scratch.append(pltpu.VMEM_SHARED((NP, CW), jnp.float32))
    scratch.append(pltpu.SemaphoreType.DMA)
    scratch.append(pltpu.SemaphoreType.DMA)
    scratch.append(pltpu.SemaphoreType.DMA)
    return pl.kernel(
        functools.partial(_sc_agg_body, d, with_cnt),
        out_type=out_type,
        mesh=mesh,
        scratch_types=scratch,
        compiler_params=pltpu.CompilerParams(use_tc_tiling_on_sc=False),
    )


def kernel(x, edge_index, W1l, b1, W1r, g1, be1, W2l, b2, W2r, g2, be2):
    src = edge_index[0].reshape(NW, NWIN, W)
    dst = edge_index[1].reshape(NW, NWIN, W)

    y1, r1 = _proj(x, W1l, W1r)
    acc1, cnt = _make_sc_agg(H1, True)(y1, src, dst)
    y2, r2 = _mid(acc1, cnt, r1, b1.reshape(1, H1), g1.reshape(1, H1),
                  be1.reshape(1, H1), W2l, W2r)
    acc2, = _make_sc_agg(H2, False)(y2, src, dst)
    return _fin(acc2, cnt, r2, b2.reshape(1, H2), g2.reshape(1, H2),
                be2.reshape(1, H2))


# speedup vs baseline: 2.4770x; 1.2201x over previous
import functools

import jax
import jax.numpy as jnp
from jax import lax
from jax.experimental import pallas as pl
from jax.experimental.pallas import tpu as pltpu
from jax.experimental.pallas import tpu_sc as plsc

N = 10000
E = 320000
D_IN = 128
H1 = 64
H2 = 32
EPS = 1e-5

NC = 2
NS = 16
NW = NC * NS
EPW = E // NW
W = 80
NWIN = EPW // W
NP = 10240
NPT = NP // NS
CW = 16
ZCH = 128


def _proj_body(x_ref, wl_ref, wr_ref, yl_ref, yr_ref):
    x = x_ref[...]
    yl_ref[...] = lax.dot_general(
        x, wl_ref[...], (((1,), (1,)), ((), ())),
        preferred_element_type=jnp.float32,
        precision=lax.Precision.HIGHEST)
    yr_ref[...] = lax.dot_general(
        x, wr_ref[...], (((1,), (1,)), ((), ())),
        preferred_element_type=jnp.float32,
        precision=lax.Precision.HIGHEST)


def _proj(x, wl, wr):
    h = wl.shape[0]
    return pl.pallas_call(
        _proj_body,
        out_shape=(
            jax.ShapeDtypeStruct((x.shape[0], h), jnp.float32),
            jax.ShapeDtypeStruct((x.shape[0], h), jnp.float32),
        ),
    )(x, wl, wr)


def _bn_relu(h, g_ref, be_ref):
    mu = jnp.mean(h, axis=0, keepdims=True)
    var = jnp.mean((h - mu) * (h - mu), axis=0, keepdims=True)
    hn = (h - mu) * lax.rsqrt(var + EPS) * g_ref[...] + be_ref[...]
    return jnp.maximum(hn, 0.0)


def _mid_body(acc_ref, cnt_ref, r_ref, b_ref, g_ref, be_ref, wl_ref, wr_ref,
              y2_ref, r2_ref):
    cnt = cnt_ref[0, 0:N, 0:1] + cnt_ref[1, 0:N, 0:1]
    agg = (acc_ref[0, 0:N] + acc_ref[1, 0:N]) / jnp.maximum(cnt, 1.0)
    h = agg + b_ref[...] + r_ref[...]
    h = _bn_relu(h, g_ref, be_ref)
    y2_ref[...] = lax.dot_general(
        h, wl_ref[...], (((1,), (1,)), ((), ())),
        preferred_element_type=jnp.float32,
        precision=lax.Precision.HIGHEST)
    r2_ref[...] = lax.dot_general(
        h, wr_ref[...], (((1,), (1,)), ((), ())),
        preferred_element_type=jnp.float32,
        precision=lax.Precision.HIGHEST)


def _mid(acc, cnt, r, b, g, be, wl, wr):
    return pl.pallas_call(
        _mid_body,
        out_shape=(
            jax.ShapeDtypeStruct((N, H2), jnp.float32),
            jax.ShapeDtypeStruct((N, H2), jnp.float32),
        ),
        compiler_params=pltpu.CompilerParams(vmem_limit_bytes=64 * 1024 * 1024),
    )(acc, cnt, r, b, g, be, wl, wr)


def _fin_body(acc_ref, cnt_ref, r_ref, b_ref, g_ref, be_ref, o_ref):
    cnt = cnt_ref[0, 0:N, 0:1] + cnt_ref[1, 0:N, 0:1]
    agg = (acc_ref[0, 0:N] + acc_ref[1, 0:N]) / jnp.maximum(cnt, 1.0)
    h = agg + b_ref[...] + r_ref[...]
    o_ref[...] = _bn_relu(h, g_ref, be_ref)


def _fin(acc, cnt, r, b, g, be):
    return pl.pallas_call(
        _fin_body,
        out_shape=jax.ShapeDtypeStruct((N, H2), jnp.float32),
    )(acc, cnt, r, b, g, be)


NBUF = 4


def _sc_agg_body(d, with_cnt, *refs):
    if with_cnt:
        (y_hbm, src_hbm, dst_hbm, acc_out, cnt_out,
         idx_s, idx_d, r0, r1, r2, r3, ones_v, zer_v, zerc_v,
         acc_sh, cnt_sh, sem, s0, s1, s2, s3) = refs
    else:
        (y_hbm, src_hbm, dst_hbm, acc_out,
         idx_s, idx_d, r0, r1, r2, r3, zer_v, acc_sh,
         sem, s0, s1, s2, s3) = refs
    rows = (r0, r1, r2, r3)
    sems = (s0, s1, s2, s3)

    c = lax.axis_index("c")
    s = lax.axis_index("s")
    wid = s * NC + c

    @pl.loop(0, ZCH)
    def _(i):
        for k in range(d // 16):
            zer_v[i, pl.ds(16 * k, 16)] = jnp.zeros((16,), jnp.float32)
        if with_cnt:
            zerc_v[i] = jnp.zeros((16,), jnp.float32)

    if with_cnt:
        @pl.loop(0, W)
        def _(i):
            ones_v[i] = jnp.ones((16,), jnp.float32)

    cp_s = pltpu.async_copy(src_hbm.at[wid], idx_s, sem)
    cp_d = pltpu.async_copy(dst_hbm.at[wid], idx_d, sem)

    @pl.loop(0, NPT // ZCH)
    def _(k):
        base = s * NPT + k * ZCH
        pltpu.sync_copy(zer_v, acc_sh.at[pl.ds(base, ZCH)])
        if with_cnt:
            pltpu.sync_copy(zerc_v, cnt_sh.at[pl.ds(base, ZCH)])

    cp_s.wait()
    cp_d.wait()
    plsc.subcore_barrier()

    for b in range(NBUF):
        pltpu.async_copy(y_hbm.at[idx_s.at[b]], rows[b], sems[b])

    @pl.loop(0, NWIN - NWIN % NBUF, step=NBUF)
    def _(j):
        for b in range(NBUF):
            if with_cnt:
                pltpu.sync_copy(ones_v, cnt_sh.at[idx_d.at[j + b]], add=True)
            pltpu.make_async_copy(y_hbm.at[pl.ds(0, W)], rows[b],
                                  sems[b]).wait()
            pltpu.sync_copy(rows[b], acc_sh.at[idx_d.at[j + b]], add=True)

            @pl.when(j + NBUF + b < NWIN)
            def _():
                pltpu.async_copy(y_hbm.at[idx_s.at[j + NBUF + b]], rows[b],
                                 sems[b])

    for b in range(NWIN % NBUF):
        t = NWIN - NWIN % NBUF + b
        if with_cnt:
            pltpu.sync_copy(ones_v, cnt_sh.at[idx_d.at[t]], add=True)
        pltpu.make_async_copy(y_hbm.at[pl.ds(0, W)], rows[b], sems[b]).wait()
        pltpu.sync_copy(rows[b], acc_sh.at[idx_d.at[t]], add=True)

    plsc.subcore_barrier()

    pltpu.sync_copy(acc_sh.at[pl.ds(s * NPT, NPT)],
                    acc_out.at[c, pl.ds(s * NPT, NPT)])
    if with_cnt:
        pltpu.sync_copy(cnt_sh.at[pl.ds(s * NPT, NPT)],
                        cnt_out.at[c, pl.ds(s * NPT, NPT)])


def _make_sc_agg(d, with_cnt):
    mesh = plsc.VectorSubcoreMesh(core_axis_name="c", subcore_axis_name="s")
    out_type = [jax.ShapeDtypeStruct((NC, NP, d), jnp.float32)]
    scratch = [
        pltpu.VMEM((NWIN, W), jnp.int32),
        pltpu.VMEM((NWIN, W), jnp.int32),
    ] + [pltpu.VMEM((W, d), jnp.float32) for _ in range(NBUF)]
    if with_cnt:
        out_type.append(jax.ShapeDtypeStruct((NC, NP, CW), jnp.float32))
        scratch.append(pltpu.VMEM((W, CW), jnp.float32))
    scratch.append(pltpu.VMEM((ZCH, d), jnp.float32))
    if with_cnt:
        scratch.append(pltpu.VMEM((ZCH, CW), jnp.float32))
    scratch.append(pltpu.VMEM_SHARED((NP, d), jnp.float32))
    if with_cnt:
        scratch.append(pltpu.VMEM_SHARED((NP, CW), jnp.float32))
    for _ in range(NBUF + 1):
        scratch.append(pltpu.SemaphoreType.DMA)
    return pl.kernel(
        functools.partial(_sc_agg_body, d, with_cnt),
        out_type=out_type,
        mesh=mesh,
        scratch_types=scratch,
        compiler_params=pltpu.CompilerParams(use_tc_tiling_on_sc=False),
    )


def kernel(x, edge_index, W1l, b1, W1r, g1, be1, W2l, b2, W2r, g2, be2):
    src = edge_index[0].reshape(NW, NWIN, W)
    dst = edge_index[1].reshape(NW, NWIN, W)

    y1, r1 = _proj(x, W1l, W1r)
    acc1, cnt = _make_sc_agg(H1, True)(y1, src, dst)
    y2, r2 = _mid(acc1, cnt, r1, b1.reshape(1, H1), g1.reshape(1, H1),
                  be1.reshape(1, H1), W2l, W2r)
    acc2, = _make_sc_agg(H2, False)(y2, src, dst)
    return _fin(acc2, cnt, r2, b2.reshape(1, H2), g2.reshape(1, H2),
                be2.reshape(1, H2))


# speedup vs baseline: 2.5560x; 1.0319x over previous
import functools

import jax
import jax.numpy as jnp
from jax import lax
from jax.experimental import pallas as pl
from jax.experimental.pallas import tpu as pltpu
from jax.experimental.pallas import tpu_sc as plsc

N = 10000
E = 320000
D_IN = 128
H1 = 64
H2 = 32
EPS = 1e-5

NC = 2
NS = 16
NW = NC * NS
EPW = E // NW
W = 80
NWIN = EPW // W
NP = 10240
NPT = NP // NS
CW = 16
ZCH = 128


def _proj_body(x_ref, wl_ref, wr_ref, yl_ref, yr_ref):
    x = x_ref[...]
    yl_ref[...] = lax.dot_general(
        x, wl_ref[...], (((1,), (1,)), ((), ())),
        preferred_element_type=jnp.float32,
        precision=lax.Precision.HIGHEST)
    yr_ref[...] = lax.dot_general(
        x, wr_ref[...], (((1,), (1,)), ((), ())),
        preferred_element_type=jnp.float32,
        precision=lax.Precision.HIGHEST)


def _proj(x, wl, wr):
    h = wl.shape[0]
    return pl.pallas_call(
        _proj_body,
        out_shape=(
            jax.ShapeDtypeStruct((x.shape[0], h), jnp.float32),
            jax.ShapeDtypeStruct((x.shape[0], h), jnp.float32),
        ),
    )(x, wl, wr)


def _bn_relu(h, g_ref, be_ref):
    mu = jnp.mean(h, axis=0, keepdims=True)
    var = jnp.mean((h - mu) * (h - mu), axis=0, keepdims=True)
    hn = (h - mu) * lax.rsqrt(var + EPS) * g_ref[...] + be_ref[...]
    return jnp.maximum(hn, 0.0)


def _mid_body(acc_ref, cnt_ref, r_ref, b_ref, g_ref, be_ref, wl_ref, wr_ref,
              y2_ref, r2_ref):
    cnt = cnt_ref[0, 0:N, 0:1] + cnt_ref[1, 0:N, 0:1]
    agg = (acc_ref[0, 0:N] + acc_ref[1, 0:N]) / jnp.maximum(cnt, 1.0)
    h = agg + b_ref[...] + r_ref[...]
    h = _bn_relu(h, g_ref, be_ref)
    y2_ref[...] = lax.dot_general(
        h, wl_ref[...], (((1,), (1,)), ((), ())),
        preferred_element_type=jnp.float32,
        precision=lax.Precision.HIGHEST)
    r2_ref[...] = lax.dot_general(
        h, wr_ref[...], (((1,), (1,)), ((), ())),
        preferred_element_type=jnp.float32,
        precision=lax.Precision.HIGHEST)


def _mid(acc, cnt, r, b, g, be, wl, wr):
    return pl.pallas_call(
        _mid_body,
        out_shape=(
            jax.ShapeDtypeStruct((N, H2), jnp.float32),
            jax.ShapeDtypeStruct((N, H2), jnp.float32),
        ),
        compiler_params=pltpu.CompilerParams(vmem_limit_bytes=64 * 1024 * 1024),
    )(acc, cnt, r, b, g, be, wl, wr)


def _fin_body(acc_ref, cnt_ref, r_ref, b_ref, g_ref, be_ref, o_ref):
    cnt = cnt_ref[0, 0:N, 0:1] + cnt_ref[1, 0:N, 0:1]
    agg = (acc_ref[0, 0:N] + acc_ref[1, 0:N]) / jnp.maximum(cnt, 1.0)
    h = agg + b_ref[...] + r_ref[...]
    o_ref[...] = _bn_relu(h, g_ref, be_ref)


def _fin(acc, cnt, r, b, g, be):
    return pl.pallas_call(
        _fin_body,
        out_shape=jax.ShapeDtypeStruct((N, H2), jnp.float32),
    )(acc, cnt, r, b, g, be)


NBUF = 8


def _sc_agg_body(d, with_cnt, *refs):
    refs = list(refs)
    if with_cnt:
        y_hbm, src_hbm, dst_hbm, acc_out, cnt_out, idx_s, idx_d = refs[:7]
        rows = refs[7:7 + NBUF]
        ones_v, zer_v, zerc_v, acc_sh, cnt_sh, sem = refs[7 + NBUF:
                                                          13 + NBUF]
        sems = refs[13 + NBUF:]
    else:
        cnt_out = ones_v = zerc_v = cnt_sh = None
        y_hbm, src_hbm, dst_hbm, acc_out, idx_s, idx_d = refs[:6]
        rows = refs[6:6 + NBUF]
        zer_v, acc_sh, sem = refs[6 + NBUF:9 + NBUF]
        sems = refs[9 + NBUF:]

    c = lax.axis_index("c")
    s = lax.axis_index("s")
    wid = s * NC + c

    @pl.loop(0, ZCH)
    def _(i):
        for k in range(d // 16):
            zer_v[i, pl.ds(16 * k, 16)] = jnp.zeros((16,), jnp.float32)
        if with_cnt:
            zerc_v[i] = jnp.zeros((16,), jnp.float32)

    if with_cnt:
        @pl.loop(0, W)
        def _(i):
            ones_v[i] = jnp.ones((16,), jnp.float32)

    cp_s = pltpu.async_copy(src_hbm.at[wid], idx_s, sem)
    cp_d = pltpu.async_copy(dst_hbm.at[wid], idx_d, sem)

    @pl.loop(0, NPT // ZCH)
    def _(k):
        base = s * NPT + k * ZCH
        pltpu.sync_copy(zer_v, acc_sh.at[pl.ds(base, ZCH)])
        if with_cnt:
            pltpu.sync_copy(zerc_v, cnt_sh.at[pl.ds(base, ZCH)])

    cp_s.wait()
    cp_d.wait()
    plsc.subcore_barrier()

    for b in range(NBUF):
        pltpu.async_copy(y_hbm.at[idx_s.at[b]], rows[b], sems[b])

    @pl.loop(0, NWIN - NWIN % NBUF, step=NBUF)
    def _(j):
        for b in range(NBUF):
            if with_cnt:
                pltpu.sync_copy(ones_v, cnt_sh.at[idx_d.at[j + b]], add=True)
            pltpu.make_async_copy(y_hbm.at[pl.ds(0, W)], rows[b],
                                  sems[b]).wait()
            pltpu.sync_copy(rows[b], acc_sh.at[idx_d.at[j + b]], add=True)

            @pl.when(j + NBUF + b < NWIN)
            def _():
                pltpu.async_copy(y_hbm.at[idx_s.at[j + NBUF + b]], rows[b],
                                 sems[b])

    for b in range(NWIN % NBUF):
        t = NWIN - NWIN % NBUF + b
        if with_cnt:
            pltpu.sync_copy(ones_v, cnt_sh.at[idx_d.at[t]], add=True)
        pltpu.make_async_copy(y_hbm.at[pl.ds(0, W)], rows[b], sems[b]).wait()
        pltpu.sync_copy(rows[b], acc_sh.at[idx_d.at[t]], add=True)

    plsc.subcore_barrier()

    pltpu.sync_copy(acc_sh.at[pl.ds(s * NPT, NPT)],
                    acc_out.at[c, pl.ds(s * NPT, NPT)])
    if with_cnt:
        pltpu.sync_copy(cnt_sh.at[pl.ds(s * NPT, NPT)],
                        cnt_out.at[c, pl.ds(s * NPT, NPT)])


def _make_sc_agg(d, with_cnt):
    mesh = plsc.VectorSubcoreMesh(core_axis_name="c", subcore_axis_name="s")
    out_type = [jax.ShapeDtypeStruct((NC, NP, d), jnp.float32)]
    scratch = [
        pltpu.VMEM((NWIN, W), jnp.int32),
        pltpu.VMEM((NWIN, W), jnp.int32),
    ] + [pltpu.VMEM((W, d), jnp.float32) for _ in range(NBUF)]
    if with_cnt:
        out_type.append(jax.ShapeDtypeStruct((NC, NP, CW), jnp.float32))
        scratch.append(pltpu.VMEM((W, CW), jnp.float32))
    scratch.append(pltpu.VMEM((ZCH, d), jnp.float32))
    if with_cnt:
        scratch.append(pltpu.VMEM((ZCH, CW), jnp.float32))
    scratch.append(pltpu.VMEM_SHARED((NP, d), jnp.float32))
    if with_cnt:
        scratch.append(pltpu.VMEM_SHARED((NP, CW), jnp.float32))
    for _ in range(NBUF + 1):
        scratch.append(pltpu.SemaphoreType.DMA)
    return pl.kernel(
        functools.partial(_sc_agg_body, d, with_cnt),
        out_type=out_type,
        mesh=mesh,
        scratch_types=scratch,
        compiler_params=pltpu.CompilerParams(use_tc_tiling_on_sc=False),
    )


def kernel(x, edge_index, W1l, b1, W1r, g1, be1, W2l, b2, W2r, g2, be2):
    src = edge_index[0].reshape(NW, NWIN, W)
    dst = edge_index[1].reshape(NW, NWIN, W)

    y1, r1 = _proj(x, W1l, W1r)
    acc1, cnt = _make_sc_agg(H1, True)(y1, src, dst)
    y2, r2 = _mid(acc1, cnt, r1, b1.reshape(1, H1), g1.reshape(1, H1),
                  be1.reshape(1, H1), W2l, W2r)
    acc2, = _make_sc_agg(H2, False)(y2, src, dst)
    return _fin(acc2, cnt, r2, b2.reshape(1, H2), g2.reshape(1, H2),
                be2.reshape(1, H2))
